# baseline (speedup 1.0000x reference)
import dataclasses

import jax
import jax.numpy as jnp
from jax import lax
from jax.experimental import pallas as pl
from jax.experimental.pallas import tpu as pltpu
from jax.experimental.pallas import tpu_sc as plsc

N = 10000
E = 320000
IN = 128
HID = 128
OUT = 128
EF = 16
CD = 3

NC = 2
NS = 16
NW = NC * NS
LN = 16
CH = 80
NROW = E // CH

S = 5
ES = E // S
PER_S = ES // NW
NCH_S = PER_S // CH
NPAIR = (NCH_S - 1) // 2

NPAD = 10240
ZROWS = NPAD // NS

NB = 1000
EB = 4000


def _sc_params():
    cp = pltpu.CompilerParams()
    if "needs_layout_passes" in pltpu.CompilerParams.__dataclass_fields__:
        cp = dataclasses.replace(cp, needs_layout_passes=False)
    return cp


def _silu(x):
    return x * (1.0 / (1.0 + jnp.exp(-x)))


def _tables_body(nf_ref, w1a_ref, w1b_ref, t1_ref, t2_ref):
    x = nf_ref[...]
    t1_ref[...] = jnp.dot(x, w1a_ref[...], preferred_element_type=jnp.float32)
    t2_ref[...] = jnp.dot(x, w1b_ref[...], preferred_element_type=jnp.float32)


def _make_tables(node_feat, w1a, w1b):
    grid = N // NB
    return pl.pallas_call(
        _tables_body,
        grid=(grid,),
        in_specs=[
            pl.BlockSpec((NB, IN), lambda i: (i, 0)),
            pl.BlockSpec((IN, HID), lambda i: (0, 0)),
            pl.BlockSpec((IN, HID), lambda i: (0, 0)),
        ],
        out_specs=[
            pl.BlockSpec((NB, HID), lambda i: (i, 0)),
            pl.BlockSpec((NB, HID), lambda i: (i, 0)),
        ],
        out_shape=[
            jax.ShapeDtypeStruct((N, HID), jnp.float32),
            jax.ShapeDtypeStruct((N, HID), jnp.float32),
        ],
    )(node_feat, w1a, w1b)


def _make_gather_kernel(sbase):
    def _gather_kernel(t1_hbm, t2_hbm, src_hbm, dst_hbm, cflat_hbm,
                       a_hbm, b_hbm, r_hbm,
                       si0, di0, ab0, bb0, rb0, si1, di1, ab1, bb1, rb1,
                       cflat, sa0, sb0, sa1, sb1, sw0, sw1):
        wid = lax.axis_index("s") * NC + lax.axis_index("c")
        base = wid * PER_S

        pltpu.sync_copy(cflat_hbm, cflat)

        col0 = jnp.zeros((LN,), jnp.int32)

        def start(j, si, di, ab, bb, sa, sb):
            goff = sbase + base + j * CH
            pltpu.sync_copy(src_hbm.at[pl.ds(goff, CH)], si)
            pltpu.sync_copy(dst_hbm.at[pl.ds(goff, CH)], di)
            pltpu.async_copy(t1_hbm.at[si], ab, sa)
            pltpu.async_copy(t2_hbm.at[di], bb, sb)

        def radial(si, di, rb):
            for k in range(CH // LN):
                ivs = si[pl.ds(k * LN, LN)]
                ivd = di[pl.ds(k * LN, LN)]
                dx = plsc.load_gather(cflat, [ivs]) \
                    - plsc.load_gather(cflat, [ivd])
                dy = plsc.load_gather(cflat, [ivs + N]) \
                    - plsc.load_gather(cflat, [ivd + N])
                dz = plsc.load_gather(cflat, [ivs + 2 * N]) \
                    - plsc.load_gather(cflat, [ivd + 2 * N])
                rows = lax.iota(jnp.int32, LN) + (k * LN)
                plsc.store_scatter(rb, [rows, col0],
                                   dx * dx + dy * dy + dz * dz)

        def process(j, si, di, ab, bb, rb, sa, sb, sw):
            radial(si, di, rb)
            pltpu.make_async_copy(t1_hbm.at[si], ab, sa).wait()
            pltpu.make_async_copy(t2_hbm.at[di], bb, sb).wait()
            off = base + j * CH
            pltpu.async_copy(ab, a_hbm.at[pl.ds(off, CH)], sw)
            pltpu.async_copy(bb, b_hbm.at[pl.ds(off, CH)], sw)
            pltpu.async_copy(rb, r_hbm.at[pl.ds(off, CH)], sw)

        def wait_wb(j, ab, bb, rb, sw):
            off = base + j * CH
            pltpu.make_async_copy(ab, a_hbm.at[pl.ds(off, CH)], sw).wait()
            pltpu.make_async_copy(bb, b_hbm.at[pl.ds(off, CH)], sw).wait()
            pltpu.make_async_copy(rb, r_hbm.at[pl.ds(off, CH)], sw).wait()

        start(0, si0, di0, ab0, bb0, sa0, sb0)
        start(1, si1, di1, ab1, bb1, sa1, sb1)

        @pl.loop(0, NPAIR)
        def _(i):
            j0 = 2 * i
            process(j0, si0, di0, ab0, bb0, rb0, sa0, sb0, sw0)
            process(j0 + 1, si1, di1, ab1, bb1, rb1, sa1, sb1, sw1)
            wait_wb(j0, ab0, bb0, rb0, sw0)
            start(j0 + 2, si0, di0, ab0, bb0, sa0, sb0)
            wait_wb(j0 + 1, ab1, bb1, rb1, sw1)

            @pl.when(j0 + 3 < NCH_S)
            def _():
                start(j0 + 3, si1, di1, ab1, bb1, sa1, sb1)

        process(NCH_S - 1, si0, di0, ab0, bb0, rb0, sa0, sb0, sw0)
        wait_wb(NCH_S - 1, ab0, bb0, rb0, sw0)

    return _gather_kernel


def _gather(sc, t1, t2, src, dst, cflat):
    mesh = plsc.VectorSubcoreMesh(core_axis_name="c", subcore_axis_name="s")
    f = pl.kernel(
        _make_gather_kernel(sc * ES),
        out_type=[
            jax.ShapeDtypeStruct((ES, HID), jnp.float32),
            jax.ShapeDtypeStruct((ES, HID), jnp.float32),
            jax.ShapeDtypeStruct((ES, LN), jnp.float32),
        ],
        mesh=mesh,
        compiler_params=_sc_params(),
        scratch_types=[
            pltpu.VMEM((CH,), jnp.int32),
            pltpu.VMEM((CH,), jnp.int32),
            pltpu.VMEM((CH, HID), jnp.float32),
            pltpu.VMEM((CH, HID), jnp.float32),
            pltpu.VMEM((CH, LN), jnp.float32),
            pltpu.VMEM((CH,), jnp.int32),
            pltpu.VMEM((CH,), jnp.int32),
            pltpu.VMEM((CH, HID), jnp.float32),
            pltpu.VMEM((CH, HID), jnp.float32),
            pltpu.VMEM((CH, LN), jnp.float32),
            pltpu.VMEM((3 * N,), jnp.float32),
            pltpu.SemaphoreType.DMA,
            pltpu.SemaphoreType.DMA,
            pltpu.SemaphoreType.DMA,
            pltpu.SemaphoreType.DMA,
            pltpu.SemaphoreType.DMA,
            pltpu.SemaphoreType.DMA,
        ],
    )
    return f(t1, t2, src, dst, cflat)


def _edge_body(a_ref, b_ref, ef_ref, r_ref, wr_ref, wef_ref, be1_ref,
               we2_ref, be2_ref, m_ref):
    radial = r_ref[...][:, :1]
    z = (a_ref[...] + b_ref[...]
         + radial * wr_ref[...]
         + jnp.dot(ef_ref[...], wef_ref[...],
                   preferred_element_type=jnp.float32)
         + be1_ref[...])
    h1 = _silu(z)
    y = jnp.dot(h1, we2_ref[...], preferred_element_type=jnp.float32) \
        + be2_ref[...]
    m_ref[...] = _silu(y)


def _edge_mlp(sc, a, b, ef, radial, wr, wef, be1, we2, be2):
    grid = ES // EB
    ebase = sc * (ES // EB)
    return pl.pallas_call(
        _edge_body,
        grid=(grid,),
        in_specs=[
            pl.BlockSpec((EB, HID), lambda i: (i, 0)),
            pl.BlockSpec((EB, HID), lambda i: (i, 0)),
            pl.BlockSpec((EB, EF), lambda i: (i + ebase, 0)),
            pl.BlockSpec((EB, LN), lambda i: (i, 0)),
            pl.BlockSpec((1, HID), lambda i: (0, 0)),
            pl.BlockSpec((EF, HID), lambda i: (0, 0)),
            pl.BlockSpec((1, HID), lambda i: (0, 0)),
            pl.BlockSpec((HID, HID), lambda i: (0, 0)),
            pl.BlockSpec((1, HID), lambda i: (0, 0)),
        ],
        out_specs=pl.BlockSpec((EB, HID), lambda i: (i, 0)),
        out_shape=jax.ShapeDtypeStruct((ES, HID), jnp.float32),
    )(a, b, ef, radial, wr, wef, be1, we2, be2)


def _make_scatter_kernel(sids):
    nm = len(sids)

    def _scatter_kernel(*refs):
        m_refs = refs[:nm]
        dst_hbm = refs[nm]
        zeros_hbm = refs[nm + 1]
        part_hbm = refs[nm + 2]
        didx, mb0, mb1, acc, sm0, sm1 = refs[nm + 3:]

        c = lax.axis_index("c")
        s = lax.axis_index("s")
        wid = s * NC + c
        zoff = s * ZROWS
        pltpu.sync_copy(zeros_hbm.at[pl.ds(zoff, ZROWS)],
                        acc.at[pl.ds(zoff, ZROWS)])
        plsc.subcore_barrier()

        base = wid * PER_S

        for sc, m_hbm in zip(sids, m_refs):
            def start(j, mb, sm, m_hbm=m_hbm):
                pltpu.async_copy(m_hbm.at[pl.ds(base + j * CH, CH)], mb, sm)

            def finish(j, mb, sm, m_hbm=m_hbm, sc=sc):
                goff = sc * ES + base + j * CH
                pltpu.sync_copy(dst_hbm.at[pl.ds(goff, CH)], didx)
                pltpu.make_async_copy(
                    m_hbm.at[pl.ds(base + j * CH, CH)], mb, sm).wait()
                pltpu.sync_copy(mb, acc.at[didx], add=True)

            start(0, mb0, sm0)

            @pl.loop(0, NPAIR)
            def _(i, start=start, finish=finish):
                j0 = 2 * i
                start(j0 + 1, mb1, sm1)
                finish(j0, mb0, sm0)
                start(j0 + 2, mb0, sm0)
                finish(j0 + 1, mb1, sm1)

            finish(NCH_S - 1, mb0, sm0)

        plsc.subcore_barrier()
        pltpu.sync_copy(acc.at[pl.ds(zoff, ZROWS)],
                        part_hbm.at[pl.ds(c * NPAD + zoff, ZROWS)])

    return _scatter_kernel


def _scatter(ms, sids, dst, zeros):
    mesh = plsc.VectorSubcoreMesh(core_axis_name="c", subcore_axis_name="s")
    f = pl.kernel(
        _make_scatter_kernel(sids),
        out_type=jax.ShapeDtypeStruct((NC * NPAD, HID), jnp.float32),
        mesh=mesh,
        compiler_params=_sc_params(),
        scratch_types=[
            pltpu.VMEM((CH,), jnp.int32),
            pltpu.VMEM((CH, HID), jnp.float32),
            pltpu.VMEM((CH, HID), jnp.float32),
            pltpu.VMEM_SHARED((NPAD, HID), jnp.float32),
            pltpu.SemaphoreType.DMA,
            pltpu.SemaphoreType.DMA,
        ],
    )
    return f(*ms, dst, zeros)


def _node_body(nf_ref, p0_ref, p1_ref, p2_ref, p3_ref, wn1a_ref, wn1b_ref,
               bn1_ref, wn2_ref, bn2_ref, o_ref):
    hn = (p0_ref[...] + p1_ref[...]) + (p2_ref[...] + p3_ref[...])
    z = (jnp.dot(nf_ref[...], wn1a_ref[...],
                 preferred_element_type=jnp.float32)
         + jnp.dot(hn, wn1b_ref[...], preferred_element_type=jnp.float32)
         + bn1_ref[...])
    h1 = _silu(z)
    o_ref[...] = jnp.dot(h1, wn2_ref[...],
                         preferred_element_type=jnp.float32) + bn2_ref[...]


def _node_mlp(nf, p0, p1, p2, p3, wn1a, wn1b, bn1, wn2, bn2):
    grid = N // NB
    return pl.pallas_call(
        _node_body,
        grid=(grid,),
        in_specs=[
            pl.BlockSpec((NB, IN), lambda i: (i, 0)),
            pl.BlockSpec((NB, HID), lambda i: (i, 0)),
            pl.BlockSpec((NB, HID), lambda i: (i, 0)),
            pl.BlockSpec((NB, HID), lambda i: (i, 0)),
            pl.BlockSpec((NB, HID), lambda i: (i, 0)),
            pl.BlockSpec((IN, HID), lambda i: (0, 0)),
            pl.BlockSpec((HID, HID), lambda i: (0, 0)),
            pl.BlockSpec((1, HID), lambda i: (0, 0)),
            pl.BlockSpec((HID, OUT), lambda i: (0, 0)),
            pl.BlockSpec((1, OUT), lambda i: (0, 0)),
        ],
        out_specs=pl.BlockSpec((NB, OUT), lambda i: (i, 0)),
        out_shape=jax.ShapeDtypeStruct((N, OUT), jnp.float32),
    )(nf, p0, p1, p2, p3, wn1a, wn1b, bn1, wn2, bn2)


def kernel(node_feat, coord, edge_feat, edge_index, We1, be1, We2, be2,
           Wn1, bn1, Wn2, bn2):
    src = edge_index[0]
    dst = edge_index[1]
    cflat = coord.T.reshape(-1)

    w1a = We1[:IN]
    w1b = We1[IN:2 * IN]
    wr = We1[2 * IN:2 * IN + 1]
    wef = We1[2 * IN + 1:]
    be1r = be1.reshape(1, -1)
    be2r = be2.reshape(1, -1)

    t1, t2 = _make_tables(node_feat, w1a, w1b)

    ms = []
    for sc in range(S):
        a, b, radial = _gather(sc, t1, t2, src, dst, cflat)
        m = _edge_mlp(sc, a, b, edge_feat, radial,
                      wr, wef, be1r, We2, be2r)
        ms.append(m)

    zeros = jnp.zeros((NPAD, HID), jnp.float32)
    part_a = _scatter(ms[:3], (0, 1, 2), dst, zeros)
    part_b = _scatter(ms[3:], (3, 4), dst, zeros)
    p0 = part_a[:N]
    p1 = part_a[NPAD:NPAD + N]
    p2 = part_b[:N]
    p3 = part_b[NPAD:NPAD + N]
    return _node_mlp(node_feat, p0, p1, p2, p3, Wn1[:IN], Wn1[IN:],
                     bn1.reshape(1, -1), Wn2, bn2.reshape(1, -1))


# speedup vs baseline: 1.0075x; 1.0075x over previous
import dataclasses

import jax
import jax.numpy as jnp
from jax import lax
from jax.experimental import pallas as pl
from jax.experimental.pallas import tpu as pltpu
from jax.experimental.pallas import tpu_sc as plsc

N = 10000
E = 320000
IN = 128
HID = 128
OUT = 128
EF = 16
CD = 3

NC = 2
NS = 16
NW = NC * NS
LN = 16
CH = 80
NROW = E // CH

S = 5
ES = E // S
PER_S = ES // NW
NCH_S = PER_S // CH
NPAIR = (NCH_S - 1) // 2

NPAD = 10240
ZROWS = NPAD // NS

NB = 1000
EB = 4000


def _sc_params():
    cp = pltpu.CompilerParams()
    if "needs_layout_passes" in pltpu.CompilerParams.__dataclass_fields__:
        cp = dataclasses.replace(cp, needs_layout_passes=False)
    return cp


def _silu(x):
    return x * (1.0 / (1.0 + jnp.exp(-x)))


def _tables_body(nf_ref, w1a_ref, w1b_ref, t1_ref, t2_ref):
    x = nf_ref[...]
    t1_ref[...] = jnp.dot(x, w1a_ref[...], preferred_element_type=jnp.float32)
    t2_ref[...] = jnp.dot(x, w1b_ref[...], preferred_element_type=jnp.float32)


def _make_tables(node_feat, w1a, w1b):
    grid = N // NB
    return pl.pallas_call(
        _tables_body,
        grid=(grid,),
        in_specs=[
            pl.BlockSpec((NB, IN), lambda i: (i, 0)),
            pl.BlockSpec((IN, HID), lambda i: (0, 0)),
            pl.BlockSpec((IN, HID), lambda i: (0, 0)),
        ],
        out_specs=[
            pl.BlockSpec((NB, HID), lambda i: (i, 0)),
            pl.BlockSpec((NB, HID), lambda i: (i, 0)),
        ],
        out_shape=[
            jax.ShapeDtypeStruct((N, HID), jnp.float32),
            jax.ShapeDtypeStruct((N, HID), jnp.float32),
        ],
    )(node_feat, w1a, w1b)


def _make_gather_kernel(sbase):
    def _gather_kernel(t1_hbm, t2_hbm, src_hbm, dst_hbm, cflat_hbm,
                       a_hbm, b_hbm, r_hbm,
                       si0, di0, ab0, bb0, rb0, si1, di1, ab1, bb1, rb1,
                       cflat, sa0, sb0, sa1, sb1, sw0, sw1):
        wid = lax.axis_index("s") * NC + lax.axis_index("c")
        base = wid * PER_S

        pltpu.sync_copy(cflat_hbm, cflat)

        col0 = jnp.zeros((LN,), jnp.int32)

        def start(j, si, di, ab, bb, sa, sb):
            goff = sbase + base + j * CH
            pltpu.sync_copy(src_hbm.at[pl.ds(goff, CH)], si)
            pltpu.sync_copy(dst_hbm.at[pl.ds(goff, CH)], di)
            pltpu.async_copy(t1_hbm.at[si], ab, sa)
            pltpu.async_copy(t2_hbm.at[di], bb, sb)

        def radial(si, di, rb):
            for k in range(CH // LN):
                ivs = si[pl.ds(k * LN, LN)]
                ivd = di[pl.ds(k * LN, LN)]
                dx = plsc.load_gather(cflat, [ivs]) \
                    - plsc.load_gather(cflat, [ivd])
                dy = plsc.load_gather(cflat, [ivs + N]) \
                    - plsc.load_gather(cflat, [ivd + N])
                dz = plsc.load_gather(cflat, [ivs + 2 * N]) \
                    - plsc.load_gather(cflat, [ivd + 2 * N])
                rows = lax.iota(jnp.int32, LN) + (k * LN)
                plsc.store_scatter(rb, [rows, col0],
                                   dx * dx + dy * dy + dz * dz)

        def process(j, si, di, ab, bb, rb, sa, sb, sw):
            radial(si, di, rb)
            pltpu.make_async_copy(t1_hbm.at[si], ab, sa).wait()
            pltpu.make_async_copy(t2_hbm.at[di], bb, sb).wait()
            off = base + j * CH
            pltpu.async_copy(ab, a_hbm.at[pl.ds(off, CH)], sw)
            pltpu.async_copy(bb, b_hbm.at[pl.ds(off, CH)], sw)
            pltpu.async_copy(rb, r_hbm.at[pl.ds(off, CH)], sw)

        def wait_wb(j, ab, bb, rb, sw):
            off = base + j * CH
            pltpu.make_async_copy(ab, a_hbm.at[pl.ds(off, CH)], sw).wait()
            pltpu.make_async_copy(bb, b_hbm.at[pl.ds(off, CH)], sw).wait()
            pltpu.make_async_copy(rb, r_hbm.at[pl.ds(off, CH)], sw).wait()

        start(0, si0, di0, ab0, bb0, sa0, sb0)
        start(1, si1, di1, ab1, bb1, sa1, sb1)

        @pl.loop(0, NPAIR)
        def _(i):
            j0 = 2 * i
            process(j0, si0, di0, ab0, bb0, rb0, sa0, sb0, sw0)
            wait_wb(j0, ab0, bb0, rb0, sw0)
            start(j0 + 2, si0, di0, ab0, bb0, sa0, sb0)
            process(j0 + 1, si1, di1, ab1, bb1, rb1, sa1, sb1, sw1)
            wait_wb(j0 + 1, ab1, bb1, rb1, sw1)

            @pl.when(j0 + 3 < NCH_S)
            def _():
                start(j0 + 3, si1, di1, ab1, bb1, sa1, sb1)

        process(NCH_S - 1, si0, di0, ab0, bb0, rb0, sa0, sb0, sw0)
        wait_wb(NCH_S - 1, ab0, bb0, rb0, sw0)

    return _gather_kernel


def _gather(sc, t1, t2, src, dst, cflat):
    mesh = plsc.VectorSubcoreMesh(core_axis_name="c", subcore_axis_name="s")
    f = pl.kernel(
        _make_gather_kernel(sc * ES),
        out_type=[
            jax.ShapeDtypeStruct((ES, HID), jnp.float32),
            jax.ShapeDtypeStruct((ES, HID), jnp.float32),
            jax.ShapeDtypeStruct((ES, LN), jnp.float32),
        ],
        mesh=mesh,
        compiler_params=_sc_params(),
        scratch_types=[
            pltpu.VMEM((CH,), jnp.int32),
            pltpu.VMEM((CH,), jnp.int32),
            pltpu.VMEM((CH, HID), jnp.float32),
            pltpu.VMEM((CH, HID), jnp.float32),
            pltpu.VMEM((CH, LN), jnp.float32),
            pltpu.VMEM((CH,), jnp.int32),
            pltpu.VMEM((CH,), jnp.int32),
            pltpu.VMEM((CH, HID), jnp.float32),
            pltpu.VMEM((CH, HID), jnp.float32),
            pltpu.VMEM((CH, LN), jnp.float32),
            pltpu.VMEM((3 * N,), jnp.float32),
            pltpu.SemaphoreType.DMA,
            pltpu.SemaphoreType.DMA,
            pltpu.SemaphoreType.DMA,
            pltpu.SemaphoreType.DMA,
            pltpu.SemaphoreType.DMA,
            pltpu.SemaphoreType.DMA,
        ],
    )
    return f(t1, t2, src, dst, cflat)


def _edge_body(a_ref, b_ref, ef_ref, r_ref, wr_ref, wef_ref, be1_ref,
               we2_ref, be2_ref, m_ref):
    radial = r_ref[...][:, :1]
    z = (a_ref[...] + b_ref[...]
         + radial * wr_ref[...]
         + jnp.dot(ef_ref[...], wef_ref[...],
                   preferred_element_type=jnp.float32)
         + be1_ref[...])
    h1 = _silu(z)
    y = jnp.dot(h1, we2_ref[...], preferred_element_type=jnp.float32) \
        + be2_ref[...]
    m_ref[...] = _silu(y)


def _edge_mlp(sc, a, b, ef, radial, wr, wef, be1, we2, be2):
    grid = ES // EB
    ebase = sc * (ES // EB)
    return pl.pallas_call(
        _edge_body,
        grid=(grid,),
        in_specs=[
            pl.BlockSpec((EB, HID), lambda i: (i, 0)),
            pl.BlockSpec((EB, HID), lambda i: (i, 0)),
            pl.BlockSpec((EB, EF), lambda i: (i + ebase, 0)),
            pl.BlockSpec((EB, LN), lambda i: (i, 0)),
            pl.BlockSpec((1, HID), lambda i: (0, 0)),
            pl.BlockSpec((EF, HID), lambda i: (0, 0)),
            pl.BlockSpec((1, HID), lambda i: (0, 0)),
            pl.BlockSpec((HID, HID), lambda i: (0, 0)),
            pl.BlockSpec((1, HID), lambda i: (0, 0)),
        ],
        out_specs=pl.BlockSpec((EB, HID), lambda i: (i, 0)),
        out_shape=jax.ShapeDtypeStruct((ES, HID), jnp.float32),
    )(a, b, ef, radial, wr, wef, be1, we2, be2)


def _make_scatter_kernel(sids):
    nm = len(sids)

    def _scatter_kernel(*refs):
        m_refs = refs[:nm]
        dst_hbm = refs[nm]
        zeros_hbm = refs[nm + 1]
        part_hbm = refs[nm + 2]
        didx, mb0, mb1, acc, sm0, sm1 = refs[nm + 3:]

        c = lax.axis_index("c")
        s = lax.axis_index("s")
        wid = s * NC + c
        zoff = s * ZROWS
        pltpu.sync_copy(zeros_hbm.at[pl.ds(zoff, ZROWS)],
                        acc.at[pl.ds(zoff, ZROWS)])
        plsc.subcore_barrier()

        base = wid * PER_S

        for sc, m_hbm in zip(sids, m_refs):
            def start(j, mb, sm, m_hbm=m_hbm):
                pltpu.async_copy(m_hbm.at[pl.ds(base + j * CH, CH)], mb, sm)

            def finish(j, mb, sm, m_hbm=m_hbm, sc=sc):
                goff = sc * ES + base + j * CH
                pltpu.sync_copy(dst_hbm.at[pl.ds(goff, CH)], didx)
                pltpu.make_async_copy(
                    m_hbm.at[pl.ds(base + j * CH, CH)], mb, sm).wait()
                pltpu.sync_copy(mb, acc.at[didx], add=True)

            start(0, mb0, sm0)

            @pl.loop(0, NPAIR)
            def _(i, start=start, finish=finish):
                j0 = 2 * i
                start(j0 + 1, mb1, sm1)
                finish(j0, mb0, sm0)
                start(j0 + 2, mb0, sm0)
                finish(j0 + 1, mb1, sm1)

            finish(NCH_S - 1, mb0, sm0)

        plsc.subcore_barrier()
        pltpu.sync_copy(acc.at[pl.ds(zoff, ZROWS)],
                        part_hbm.at[pl.ds(c * NPAD + zoff, ZROWS)])

    return _scatter_kernel


def _scatter(ms, sids, dst, zeros):
    mesh = plsc.VectorSubcoreMesh(core_axis_name="c", subcore_axis_name="s")
    f = pl.kernel(
        _make_scatter_kernel(sids),
        out_type=jax.ShapeDtypeStruct((NC * NPAD, HID), jnp.float32),
        mesh=mesh,
        compiler_params=_sc_params(),
        scratch_types=[
            pltpu.VMEM((CH,), jnp.int32),
            pltpu.VMEM((CH, HID), jnp.float32),
            pltpu.VMEM((CH, HID), jnp.float32),
            pltpu.VMEM_SHARED((NPAD, HID), jnp.float32),
            pltpu.SemaphoreType.DMA,
            pltpu.SemaphoreType.DMA,
        ],
    )
    return f(*ms, dst, zeros)


def _node_body(nf_ref, p0_ref, p1_ref, p2_ref, p3_ref, wn1a_ref, wn1b_ref,
               bn1_ref, wn2_ref, bn2_ref, o_ref):
    hn = (p0_ref[...] + p1_ref[...]) + (p2_ref[...] + p3_ref[...])
    z = (jnp.dot(nf_ref[...], wn1a_ref[...],
                 preferred_element_type=jnp.float32)
         + jnp.dot(hn, wn1b_ref[...], preferred_element_type=jnp.float32)
         + bn1_ref[...])
    h1 = _silu(z)
    o_ref[...] = jnp.dot(h1, wn2_ref[...],
                         preferred_element_type=jnp.float32) + bn2_ref[...]


def _node_mlp(nf, p0, p1, p2, p3, wn1a, wn1b, bn1, wn2, bn2):
    grid = N // NB
    return pl.pallas_call(
        _node_body,
        grid=(grid,),
        in_specs=[
            pl.BlockSpec((NB, IN), lambda i: (i, 0)),
            pl.BlockSpec((NB, HID), lambda i: (i, 0)),
            pl.BlockSpec((NB, HID), lambda i: (i, 0)),
            pl.BlockSpec((NB, HID), lambda i: (i, 0)),
            pl.BlockSpec((NB, HID), lambda i: (i, 0)),
            pl.BlockSpec((IN, HID), lambda i: (0, 0)),
            pl.BlockSpec((HID, HID), lambda i: (0, 0)),
            pl.BlockSpec((1, HID), lambda i: (0, 0)),
            pl.BlockSpec((HID, OUT), lambda i: (0, 0)),
            pl.BlockSpec((1, OUT), lambda i: (0, 0)),
        ],
        out_specs=pl.BlockSpec((NB, OUT), lambda i: (i, 0)),
        out_shape=jax.ShapeDtypeStruct((N, OUT), jnp.float32),
    )(nf, p0, p1, p2, p3, wn1a, wn1b, bn1, wn2, bn2)


def kernel(node_feat, coord, edge_feat, edge_index, We1, be1, We2, be2,
           Wn1, bn1, Wn2, bn2):
    src = edge_index[0]
    dst = edge_index[1]
    cflat = coord.T.reshape(-1)

    w1a = We1[:IN]
    w1b = We1[IN:2 * IN]
    wr = We1[2 * IN:2 * IN + 1]
    wef = We1[2 * IN + 1:]
    be1r = be1.reshape(1, -1)
    be2r = be2.reshape(1, -1)

    t1, t2 = _make_tables(node_feat, w1a, w1b)

    ms = []
    for sc in range(S):
        a, b, radial = _gather(sc, t1, t2, src, dst, cflat)
        m = _edge_mlp(sc, a, b, edge_feat, radial,
                      wr, wef, be1r, We2, be2r)
        ms.append(m)

    zeros = jnp.zeros((NPAD, HID), jnp.float32)
    part_a = _scatter(ms[:3], (0, 1, 2), dst, zeros)
    part_b = _scatter(ms[3:], (3, 4), dst, zeros)
    p0 = part_a[:N]
    p1 = part_a[NPAD:NPAD + N]
    p2 = part_b[:N]
    p3 = part_b[NPAD:NPAD + N]
    return _node_mlp(node_feat, p0, p1, p2, p3, Wn1[:IN], Wn1[IN:],
                     bn1.reshape(1, -1), Wn2, bn2.reshape(1, -1))


# speedup vs baseline: 1.0205x; 1.0130x over previous
import dataclasses

import jax
import jax.numpy as jnp
from jax import lax
from jax.experimental import pallas as pl
from jax.experimental.pallas import tpu as pltpu
from jax.experimental.pallas import tpu_sc as plsc

N = 10000
E = 320000
IN = 128
HID = 128
OUT = 128
EF = 16
CD = 3

NC = 2
NS = 16
NW = NC * NS
LN = 16
CH = 80
NROW = E // CH

S = 5
ES = E // S
PER_S = ES // NW
NCH_S = PER_S // CH
NPAIR = (NCH_S - 1) // 2

NPAD = 10240
ZROWS = NPAD // NS

NB = 1000
EB = 4000


def _sc_params():
    cp = pltpu.CompilerParams()
    if "needs_layout_passes" in pltpu.CompilerParams.__dataclass_fields__:
        cp = dataclasses.replace(cp, needs_layout_passes=False)
    return cp


def _silu(x):
    return x * (1.0 / (1.0 + jnp.exp(-x)))


def _tables_body(nf_ref, w1a_ref, w1b_ref, t1_ref, t2_ref):
    x = nf_ref[...]
    t1_ref[...] = jnp.dot(x, w1a_ref[...], preferred_element_type=jnp.float32)
    t2_ref[...] = jnp.dot(x, w1b_ref[...], preferred_element_type=jnp.float32)


def _make_tables(node_feat, w1a, w1b):
    grid = N // NB
    return pl.pallas_call(
        _tables_body,
        grid=(grid,),
        in_specs=[
            pl.BlockSpec((NB, IN), lambda i: (i, 0)),
            pl.BlockSpec((IN, HID), lambda i: (0, 0)),
            pl.BlockSpec((IN, HID), lambda i: (0, 0)),
        ],
        out_specs=[
            pl.BlockSpec((NB, HID), lambda i: (i, 0)),
            pl.BlockSpec((NB, HID), lambda i: (i, 0)),
        ],
        out_shape=[
            jax.ShapeDtypeStruct((N, HID), jnp.float32),
            jax.ShapeDtypeStruct((N, HID), jnp.float32),
        ],
    )(node_feat, w1a, w1b)


def _make_gather_kernel(sbase):
    def _gather_kernel(t1_hbm, t2_hbm, src_hbm, dst_hbm, cflat_hbm,
                       a_hbm, b_hbm, r_hbm,
                       sall, dall, ab0, bb0, rb0, ab1, bb1, rb1,
                       cflat, sa0, sb0, sa1, sb1, sw0, sw1):
        wid = lax.axis_index("s") * NC + lax.axis_index("c")
        base = wid * PER_S

        pltpu.sync_copy(cflat_hbm, cflat)
        pltpu.sync_copy(src_hbm.at[pl.ds(sbase + base, PER_S)], sall)
        pltpu.sync_copy(dst_hbm.at[pl.ds(sbase + base, PER_S)], dall)

        col0 = jnp.zeros((LN,), jnp.int32)

        def start(j, ab, bb, sa, sb):
            pltpu.async_copy(t1_hbm.at[sall.at[pl.ds(j * CH, CH)]], ab, sa)
            pltpu.async_copy(t2_hbm.at[dall.at[pl.ds(j * CH, CH)]], bb, sb)

        def radial(j, rb):
            for k in range(CH // LN):
                ivs = sall[pl.ds(j * CH + k * LN, LN)]
                ivd = dall[pl.ds(j * CH + k * LN, LN)]
                dx = plsc.load_gather(cflat, [ivs]) \
                    - plsc.load_gather(cflat, [ivd])
                dy = plsc.load_gather(cflat, [ivs + N]) \
                    - plsc.load_gather(cflat, [ivd + N])
                dz = plsc.load_gather(cflat, [ivs + 2 * N]) \
                    - plsc.load_gather(cflat, [ivd + 2 * N])
                rows = lax.iota(jnp.int32, LN) + (k * LN)
                plsc.store_scatter(rb, [rows, col0],
                                   dx * dx + dy * dy + dz * dz)

        def process(j, ab, bb, rb, sa, sb, sw):
            radial(j, rb)
            pltpu.make_async_copy(
                t1_hbm.at[sall.at[pl.ds(j * CH, CH)]], ab, sa).wait()
            pltpu.make_async_copy(
                t2_hbm.at[dall.at[pl.ds(j * CH, CH)]], bb, sb).wait()
            off = base + j * CH
            pltpu.async_copy(ab, a_hbm.at[pl.ds(off, CH)], sw)
            pltpu.async_copy(bb, b_hbm.at[pl.ds(off, CH)], sw)
            pltpu.async_copy(rb, r_hbm.at[pl.ds(off, CH)], sw)

        def wait_wb(j, ab, bb, rb, sw):
            off = base + j * CH
            pltpu.make_async_copy(ab, a_hbm.at[pl.ds(off, CH)], sw).wait()
            pltpu.make_async_copy(bb, b_hbm.at[pl.ds(off, CH)], sw).wait()
            pltpu.make_async_copy(rb, r_hbm.at[pl.ds(off, CH)], sw).wait()

        start(0, ab0, bb0, sa0, sb0)
        start(1, ab1, bb1, sa1, sb1)

        @pl.loop(0, NPAIR)
        def _(i):
            j0 = 2 * i
            process(j0, ab0, bb0, rb0, sa0, sb0, sw0)
            wait_wb(j0, ab0, bb0, rb0, sw0)
            start(j0 + 2, ab0, bb0, sa0, sb0)
            process(j0 + 1, ab1, bb1, rb1, sa1, sb1, sw1)
            wait_wb(j0 + 1, ab1, bb1, rb1, sw1)

            @pl.when(j0 + 3 < NCH_S)
            def _():
                start(j0 + 3, ab1, bb1, sa1, sb1)

        process(NCH_S - 1, ab0, bb0, rb0, sa0, sb0, sw0)
        wait_wb(NCH_S - 1, ab0, bb0, rb0, sw0)

    return _gather_kernel


def _gather(sc, t1, t2, src, dst, cflat):
    mesh = plsc.VectorSubcoreMesh(core_axis_name="c", subcore_axis_name="s")
    f = pl.kernel(
        _make_gather_kernel(sc * ES),
        out_type=[
            jax.ShapeDtypeStruct((ES, HID), jnp.float32),
            jax.ShapeDtypeStruct((ES, HID), jnp.float32),
            jax.ShapeDtypeStruct((ES, LN), jnp.float32),
        ],
        mesh=mesh,
        compiler_params=_sc_params(),
        scratch_types=[
            pltpu.VMEM((PER_S,), jnp.int32),
            pltpu.VMEM((PER_S,), jnp.int32),
            pltpu.VMEM((CH, HID), jnp.float32),
            pltpu.VMEM((CH, HID), jnp.float32),
            pltpu.VMEM((CH, LN), jnp.float32),
            pltpu.VMEM((CH, HID), jnp.float32),
            pltpu.VMEM((CH, HID), jnp.float32),
            pltpu.VMEM((CH, LN), jnp.float32),
            pltpu.VMEM((3 * N,), jnp.float32),
            pltpu.SemaphoreType.DMA,
            pltpu.SemaphoreType.DMA,
            pltpu.SemaphoreType.DMA,
            pltpu.SemaphoreType.DMA,
            pltpu.SemaphoreType.DMA,
            pltpu.SemaphoreType.DMA,
        ],
    )
    return f(t1, t2, src, dst, cflat)


def _edge_body(a_ref, b_ref, ef_ref, r_ref, wr_ref, wef_ref, be1_ref,
               we2_ref, be2_ref, m_ref):
    radial = r_ref[...][:, :1]
    z = (a_ref[...] + b_ref[...]
         + radial * wr_ref[...]
         + jnp.dot(ef_ref[...], wef_ref[...],
                   preferred_element_type=jnp.float32)
         + be1_ref[...])
    h1 = _silu(z)
    y = jnp.dot(h1, we2_ref[...], preferred_element_type=jnp.float32) \
        + be2_ref[...]
    m_ref[...] = _silu(y)


def _edge_mlp(sc, a, b, ef, radial, wr, wef, be1, we2, be2):
    grid = ES // EB
    ebase = sc * (ES // EB)
    return pl.pallas_call(
        _edge_body,
        grid=(grid,),
        in_specs=[
            pl.BlockSpec((EB, HID), lambda i: (i, 0)),
            pl.BlockSpec((EB, HID), lambda i: (i, 0)),
            pl.BlockSpec((EB, EF), lambda i: (i + ebase, 0)),
            pl.BlockSpec((EB, LN), lambda i: (i, 0)),
            pl.BlockSpec((1, HID), lambda i: (0, 0)),
            pl.BlockSpec((EF, HID), lambda i: (0, 0)),
            pl.BlockSpec((1, HID), lambda i: (0, 0)),
            pl.BlockSpec((HID, HID), lambda i: (0, 0)),
            pl.BlockSpec((1, HID), lambda i: (0, 0)),
        ],
        out_specs=pl.BlockSpec((EB, HID), lambda i: (i, 0)),
        out_shape=jax.ShapeDtypeStruct((ES, HID), jnp.float32),
    )(a, b, ef, radial, wr, wef, be1, we2, be2)


def _make_scatter_kernel(sids):
    nm = len(sids)

    def _scatter_kernel(*refs):
        m_refs = refs[:nm]
        dst_hbm = refs[nm]
        zeros_hbm = refs[nm + 1]
        part_hbm = refs[nm + 2]
        didx, mb0, mb1, acc, sm0, sm1 = refs[nm + 3:]

        c = lax.axis_index("c")
        s = lax.axis_index("s")
        wid = s * NC + c
        zoff = s * ZROWS
        pltpu.sync_copy(zeros_hbm.at[pl.ds(zoff, ZROWS)],
                        acc.at[pl.ds(zoff, ZROWS)])
        plsc.subcore_barrier()

        base = wid * PER_S

        for sc, m_hbm in zip(sids, m_refs):
            def start(j, mb, sm, m_hbm=m_hbm):
                pltpu.async_copy(m_hbm.at[pl.ds(base + j * CH, CH)], mb, sm)

            def finish(j, mb, sm, m_hbm=m_hbm, sc=sc):
                goff = sc * ES + base + j * CH
                pltpu.sync_copy(dst_hbm.at[pl.ds(goff, CH)], didx)
                pltpu.make_async_copy(
                    m_hbm.at[pl.ds(base + j * CH, CH)], mb, sm).wait()
                pltpu.sync_copy(mb, acc.at[didx], add=True)

            start(0, mb0, sm0)

            @pl.loop(0, NPAIR)
            def _(i, start=start, finish=finish):
                j0 = 2 * i
                start(j0 + 1, mb1, sm1)
                finish(j0, mb0, sm0)
                start(j0 + 2, mb0, sm0)
                finish(j0 + 1, mb1, sm1)

            finish(NCH_S - 1, mb0, sm0)

        plsc.subcore_barrier()
        pltpu.sync_copy(acc.at[pl.ds(zoff, ZROWS)],
                        part_hbm.at[pl.ds(c * NPAD + zoff, ZROWS)])

    return _scatter_kernel


def _scatter(ms, sids, dst, zeros):
    mesh = plsc.VectorSubcoreMesh(core_axis_name="c", subcore_axis_name="s")
    f = pl.kernel(
        _make_scatter_kernel(sids),
        out_type=jax.ShapeDtypeStruct((NC * NPAD, HID), jnp.float32),
        mesh=mesh,
        compiler_params=_sc_params(),
        scratch_types=[
            pltpu.VMEM((CH,), jnp.int32),
            pltpu.VMEM((CH, HID), jnp.float32),
            pltpu.VMEM((CH, HID), jnp.float32),
            pltpu.VMEM_SHARED((NPAD, HID), jnp.float32),
            pltpu.SemaphoreType.DMA,
            pltpu.SemaphoreType.DMA,
        ],
    )
    return f(*ms, dst, zeros)


def _node_body(nf_ref, p0_ref, p1_ref, p2_ref, p3_ref, wn1a_ref, wn1b_ref,
               bn1_ref, wn2_ref, bn2_ref, o_ref):
    hn = (p0_ref[...] + p1_ref[...]) + (p2_ref[...] + p3_ref[...])
    z = (jnp.dot(nf_ref[...], wn1a_ref[...],
                 preferred_element_type=jnp.float32)
         + jnp.dot(hn, wn1b_ref[...], preferred_element_type=jnp.float32)
         + bn1_ref[...])
    h1 = _silu(z)
    o_ref[...] = jnp.dot(h1, wn2_ref[...],
                         preferred_element_type=jnp.float32) + bn2_ref[...]


def _node_mlp(nf, p0, p1, p2, p3, wn1a, wn1b, bn1, wn2, bn2):
    grid = N // NB
    return pl.pallas_call(
        _node_body,
        grid=(grid,),
        in_specs=[
            pl.BlockSpec((NB, IN), lambda i: (i, 0)),
            pl.BlockSpec((NB, HID), lambda i: (i, 0)),
            pl.BlockSpec((NB, HID), lambda i: (i, 0)),
            pl.BlockSpec((NB, HID), lambda i: (i, 0)),
            pl.BlockSpec((NB, HID), lambda i: (i, 0)),
            pl.BlockSpec((IN, HID), lambda i: (0, 0)),
            pl.BlockSpec((HID, HID), lambda i: (0, 0)),
            pl.BlockSpec((1, HID), lambda i: (0, 0)),
            pl.BlockSpec((HID, OUT), lambda i: (0, 0)),
            pl.BlockSpec((1, OUT), lambda i: (0, 0)),
        ],
        out_specs=pl.BlockSpec((NB, OUT), lambda i: (i, 0)),
        out_shape=jax.ShapeDtypeStruct((N, OUT), jnp.float32),
    )(nf, p0, p1, p2, p3, wn1a, wn1b, bn1, wn2, bn2)


def kernel(node_feat, coord, edge_feat, edge_index, We1, be1, We2, be2,
           Wn1, bn1, Wn2, bn2):
    src = edge_index[0]
    dst = edge_index[1]
    cflat = coord.T.reshape(-1)

    w1a = We1[:IN]
    w1b = We1[IN:2 * IN]
    wr = We1[2 * IN:2 * IN + 1]
    wef = We1[2 * IN + 1:]
    be1r = be1.reshape(1, -1)
    be2r = be2.reshape(1, -1)

    t1, t2 = _make_tables(node_feat, w1a, w1b)

    ms = []
    for sc in range(S):
        a, b, radial = _gather(sc, t1, t2, src, dst, cflat)
        m = _edge_mlp(sc, a, b, edge_feat, radial,
                      wr, wef, be1r, We2, be2r)
        ms.append(m)

    zeros = jnp.zeros((NPAD, HID), jnp.float32)
    part_a = _scatter(ms[:3], (0, 1, 2), dst, zeros)
    part_b = _scatter(ms[3:], (3, 4), dst, zeros)
    p0 = part_a[:N]
    p1 = part_a[NPAD:NPAD + N]
    p2 = part_b[:N]
    p3 = part_b[NPAD:NPAD + N]
    return _node_mlp(node_feat, p0, p1, p2, p3, Wn1[:IN], Wn1[IN:],
                     bn1.reshape(1, -1), Wn2, bn2.reshape(1, -1))


# speedup vs baseline: 1.0269x; 1.0062x over previous
import dataclasses

import jax
import jax.numpy as jnp
from jax import lax
from jax.experimental import pallas as pl
from jax.experimental.pallas import tpu as pltpu
from jax.experimental.pallas import tpu_sc as plsc

N = 10000
E = 320000
IN = 128
HID = 128
OUT = 128
EF = 16
CD = 3

NC = 2
NS = 16
NW = NC * NS
LN = 16
CH = 80
NROW = E // CH

S = 5
ES = E // S
PER_S = ES // NW
NCH_S = PER_S // CH
NPAIR = (NCH_S - 1) // 2

NPAD = 10240
ZROWS = NPAD // NS

NB = 1000
EB = 4000


def _sc_params():
    cp = pltpu.CompilerParams()
    if "needs_layout_passes" in pltpu.CompilerParams.__dataclass_fields__:
        cp = dataclasses.replace(cp, needs_layout_passes=False)
    return cp


def _silu(x):
    return x * (1.0 / (1.0 + jnp.exp(-x)))


def _tables_body(nf_ref, w1a_ref, w1b_ref, t1_ref, t2_ref):
    x = nf_ref[...]
    t1_ref[...] = jnp.dot(x, w1a_ref[...], preferred_element_type=jnp.float32)
    t2_ref[...] = jnp.dot(x, w1b_ref[...], preferred_element_type=jnp.float32)


def _make_tables(node_feat, w1a, w1b):
    grid = N // NB
    return pl.pallas_call(
        _tables_body,
        grid=(grid,),
        in_specs=[
            pl.BlockSpec((NB, IN), lambda i: (i, 0)),
            pl.BlockSpec((IN, HID), lambda i: (0, 0)),
            pl.BlockSpec((IN, HID), lambda i: (0, 0)),
        ],
        out_specs=[
            pl.BlockSpec((NB, HID), lambda i: (i, 0)),
            pl.BlockSpec((NB, HID), lambda i: (i, 0)),
        ],
        out_shape=[
            jax.ShapeDtypeStruct((N, HID), jnp.float32),
            jax.ShapeDtypeStruct((N, HID), jnp.float32),
        ],
    )(node_feat, w1a, w1b)


def _make_gather_kernel(sbase):
    def _gather_kernel(t1_hbm, t2_hbm, src_hbm, dst_hbm, cflat_hbm,
                       a_hbm, b_hbm, r_hbm,
                       sall, dall, ab0, bb0, rb0, ab1, bb1, rb1,
                       cflat, sa0, sb0, sa1, sb1, sw0, sw1):
        wid = lax.axis_index("s") * NC + lax.axis_index("c")
        base = wid * PER_S

        pltpu.sync_copy(cflat_hbm, cflat)
        pltpu.sync_copy(src_hbm.at[pl.ds(sbase + base, PER_S)], sall)
        pltpu.sync_copy(dst_hbm.at[pl.ds(sbase + base, PER_S)], dall)

        col0 = jnp.zeros((LN,), jnp.int32)

        def start(j, ab, bb, sa, sb):
            pltpu.async_copy(t1_hbm.at[sall.at[pl.ds(j * CH, CH)]], ab, sa)
            pltpu.async_copy(t2_hbm.at[dall.at[pl.ds(j * CH, CH)]], bb, sb)

        def radial(j, rb):
            for k in range(CH // LN):
                ivs = sall[pl.ds(j * CH + k * LN, LN)]
                ivd = dall[pl.ds(j * CH + k * LN, LN)]
                dx = plsc.load_gather(cflat, [ivs]) \
                    - plsc.load_gather(cflat, [ivd])
                dy = plsc.load_gather(cflat, [ivs + N]) \
                    - plsc.load_gather(cflat, [ivd + N])
                dz = plsc.load_gather(cflat, [ivs + 2 * N]) \
                    - plsc.load_gather(cflat, [ivd + 2 * N])
                rows = lax.iota(jnp.int32, LN) + (k * LN)
                plsc.store_scatter(rb, [rows, col0],
                                   dx * dx + dy * dy + dz * dz)

        def process(j, ab, bb, rb, sa, sb, sw):
            radial(j, rb)
            pltpu.make_async_copy(
                t1_hbm.at[sall.at[pl.ds(j * CH, CH)]], ab, sa).wait()
            pltpu.make_async_copy(
                t2_hbm.at[dall.at[pl.ds(j * CH, CH)]], bb, sb).wait()
            off = base + j * CH
            pltpu.async_copy(ab, a_hbm.at[pl.ds(off, CH)], sw)
            pltpu.async_copy(bb, b_hbm.at[pl.ds(off, CH)], sw)
            pltpu.async_copy(rb, r_hbm.at[pl.ds(off, CH)], sw)

        def wait_wb(j, ab, bb, rb, sw):
            off = base + j * CH
            pltpu.make_async_copy(ab, a_hbm.at[pl.ds(off, CH)], sw).wait()
            pltpu.make_async_copy(bb, b_hbm.at[pl.ds(off, CH)], sw).wait()
            pltpu.make_async_copy(rb, r_hbm.at[pl.ds(off, CH)], sw).wait()

        start(0, ab0, bb0, sa0, sb0)
        start(1, ab1, bb1, sa1, sb1)

        @pl.loop(0, NPAIR)
        def _(i):
            j0 = 2 * i
            process(j0, ab0, bb0, rb0, sa0, sb0, sw0)
            wait_wb(j0, ab0, bb0, rb0, sw0)
            start(j0 + 2, ab0, bb0, sa0, sb0)
            process(j0 + 1, ab1, bb1, rb1, sa1, sb1, sw1)
            wait_wb(j0 + 1, ab1, bb1, rb1, sw1)

            @pl.when(j0 + 3 < NCH_S)
            def _():
                start(j0 + 3, ab1, bb1, sa1, sb1)

        process(NCH_S - 1, ab0, bb0, rb0, sa0, sb0, sw0)
        wait_wb(NCH_S - 1, ab0, bb0, rb0, sw0)

    return _gather_kernel


def _gather(sc, t1, t2, src, dst, cflat):
    mesh = plsc.VectorSubcoreMesh(core_axis_name="c", subcore_axis_name="s")
    f = pl.kernel(
        _make_gather_kernel(sc * ES),
        out_type=[
            jax.ShapeDtypeStruct((ES, HID), jnp.float32),
            jax.ShapeDtypeStruct((ES, HID), jnp.float32),
            jax.ShapeDtypeStruct((ES, LN), jnp.float32),
        ],
        mesh=mesh,
        compiler_params=_sc_params(),
        scratch_types=[
            pltpu.VMEM((PER_S,), jnp.int32),
            pltpu.VMEM((PER_S,), jnp.int32),
            pltpu.VMEM((CH, HID), jnp.float32),
            pltpu.VMEM((CH, HID), jnp.float32),
            pltpu.VMEM((CH, LN), jnp.float32),
            pltpu.VMEM((CH, HID), jnp.float32),
            pltpu.VMEM((CH, HID), jnp.float32),
            pltpu.VMEM((CH, LN), jnp.float32),
            pltpu.VMEM((3 * N,), jnp.float32),
            pltpu.SemaphoreType.DMA,
            pltpu.SemaphoreType.DMA,
            pltpu.SemaphoreType.DMA,
            pltpu.SemaphoreType.DMA,
            pltpu.SemaphoreType.DMA,
            pltpu.SemaphoreType.DMA,
        ],
    )
    return f(t1, t2, src, dst, cflat)


def _edge_body(a_ref, b_ref, ef_ref, r_ref, wr_ref, wef_ref, be1_ref,
               we2_ref, be2_ref, m_ref):
    radial = r_ref[...][:, :1]
    z = (a_ref[...] + b_ref[...]
         + radial * wr_ref[...]
         + jnp.dot(ef_ref[...], wef_ref[...],
                   preferred_element_type=jnp.float32)
         + be1_ref[...])
    h1 = _silu(z)
    y = jnp.dot(h1, we2_ref[...], preferred_element_type=jnp.float32) \
        + be2_ref[...]
    m_ref[...] = _silu(y)


def _edge_mlp(sc, a, b, ef, radial, wr, wef, be1, we2, be2):
    grid = ES // EB
    ebase = sc * (ES // EB)
    return pl.pallas_call(
        _edge_body,
        grid=(grid,),
        in_specs=[
            pl.BlockSpec((EB, HID), lambda i: (i, 0)),
            pl.BlockSpec((EB, HID), lambda i: (i, 0)),
            pl.BlockSpec((EB, EF), lambda i: (i + ebase, 0)),
            pl.BlockSpec((EB, LN), lambda i: (i, 0)),
            pl.BlockSpec((1, HID), lambda i: (0, 0)),
            pl.BlockSpec((EF, HID), lambda i: (0, 0)),
            pl.BlockSpec((1, HID), lambda i: (0, 0)),
            pl.BlockSpec((HID, HID), lambda i: (0, 0)),
            pl.BlockSpec((1, HID), lambda i: (0, 0)),
        ],
        out_specs=pl.BlockSpec((EB, HID), lambda i: (i, 0)),
        out_shape=jax.ShapeDtypeStruct((ES, HID), jnp.float32),
    )(a, b, ef, radial, wr, wef, be1, we2, be2)


def _make_scatter_kernel(sids):
    nm = len(sids)

    def _scatter_kernel(*refs):
        m_refs = refs[:nm]
        dst_hbm = refs[nm]
        zeros_hbm = refs[nm + 1]
        part_hbm = refs[nm + 2]
        di0, di1, mb0, mb1, acc, sm0, sm1, sd0, sd1 = refs[nm + 3:]

        c = lax.axis_index("c")
        s = lax.axis_index("s")
        wid = s * NC + c
        zoff = s * ZROWS
        pltpu.sync_copy(zeros_hbm.at[pl.ds(zoff, ZROWS)],
                        acc.at[pl.ds(zoff, ZROWS)])
        plsc.subcore_barrier()

        base = wid * PER_S

        for sc, m_hbm in zip(sids, m_refs):
            def start(j, mb, di, sm, sd, m_hbm=m_hbm, sc=sc):
                pltpu.async_copy(m_hbm.at[pl.ds(base + j * CH, CH)], mb, sm)
                goff = sc * ES + base + j * CH
                pltpu.async_copy(dst_hbm.at[pl.ds(goff, CH)], di, sd)

            def finish(j, mb, di, sm, sd, m_hbm=m_hbm, sc=sc):
                goff = sc * ES + base + j * CH
                pltpu.make_async_copy(
                    dst_hbm.at[pl.ds(goff, CH)], di, sd).wait()
                pltpu.make_async_copy(
                    m_hbm.at[pl.ds(base + j * CH, CH)], mb, sm).wait()
                pltpu.sync_copy(mb, acc.at[di], add=True)

            start(0, mb0, di0, sm0, sd0)

            @pl.loop(0, NPAIR)
            def _(i, start=start, finish=finish):
                j0 = 2 * i
                start(j0 + 1, mb1, di1, sm1, sd1)
                finish(j0, mb0, di0, sm0, sd0)
                start(j0 + 2, mb0, di0, sm0, sd0)
                finish(j0 + 1, mb1, di1, sm1, sd1)

            finish(NCH_S - 1, mb0, di0, sm0, sd0)

        plsc.subcore_barrier()
        pltpu.sync_copy(acc.at[pl.ds(zoff, ZROWS)],
                        part_hbm.at[pl.ds(c * NPAD + zoff, ZROWS)])

    return _scatter_kernel


def _scatter(ms, sids, dst, zeros):
    mesh = plsc.VectorSubcoreMesh(core_axis_name="c", subcore_axis_name="s")
    f = pl.kernel(
        _make_scatter_kernel(sids),
        out_type=jax.ShapeDtypeStruct((NC * NPAD, HID), jnp.float32),
        mesh=mesh,
        compiler_params=_sc_params(),
        scratch_types=[
            pltpu.VMEM((CH,), jnp.int32),
            pltpu.VMEM((CH,), jnp.int32),
            pltpu.VMEM((CH, HID), jnp.float32),
            pltpu.VMEM((CH, HID), jnp.float32),
            pltpu.VMEM_SHARED((NPAD, HID), jnp.float32),
            pltpu.SemaphoreType.DMA,
            pltpu.SemaphoreType.DMA,
            pltpu.SemaphoreType.DMA,
            pltpu.SemaphoreType.DMA,
        ],
    )
    return f(*ms, dst, zeros)


def _node_body(nf_ref, p0_ref, p1_ref, p2_ref, p3_ref, wn1a_ref, wn1b_ref,
               bn1_ref, wn2_ref, bn2_ref, o_ref):
    hn = (p0_ref[...] + p1_ref[...]) + (p2_ref[...] + p3_ref[...])
    z = (jnp.dot(nf_ref[...], wn1a_ref[...],
                 preferred_element_type=jnp.float32)
         + jnp.dot(hn, wn1b_ref[...], preferred_element_type=jnp.float32)
         + bn1_ref[...])
    h1 = _silu(z)
    o_ref[...] = jnp.dot(h1, wn2_ref[...],
                         preferred_element_type=jnp.float32) + bn2_ref[...]


def _node_mlp(nf, p0, p1, p2, p3, wn1a, wn1b, bn1, wn2, bn2):
    grid = N // NB
    return pl.pallas_call(
        _node_body,
        grid=(grid,),
        in_specs=[
            pl.BlockSpec((NB, IN), lambda i: (i, 0)),
            pl.BlockSpec((NB, HID), lambda i: (i, 0)),
            pl.BlockSpec((NB, HID), lambda i: (i, 0)),
            pl.BlockSpec((NB, HID), lambda i: (i, 0)),
            pl.BlockSpec((NB, HID), lambda i: (i, 0)),
            pl.BlockSpec((IN, HID), lambda i: (0, 0)),
            pl.BlockSpec((HID, HID), lambda i: (0, 0)),
            pl.BlockSpec((1, HID), lambda i: (0, 0)),
            pl.BlockSpec((HID, OUT), lambda i: (0, 0)),
            pl.BlockSpec((1, OUT), lambda i: (0, 0)),
        ],
        out_specs=pl.BlockSpec((NB, OUT), lambda i: (i, 0)),
        out_shape=jax.ShapeDtypeStruct((N, OUT), jnp.float32),
    )(nf, p0, p1, p2, p3, wn1a, wn1b, bn1, wn2, bn2)


def kernel(node_feat, coord, edge_feat, edge_index, We1, be1, We2, be2,
           Wn1, bn1, Wn2, bn2):
    src = edge_index[0]
    dst = edge_index[1]
    cflat = coord.T.reshape(-1)

    w1a = We1[:IN]
    w1b = We1[IN:2 * IN]
    wr = We1[2 * IN:2 * IN + 1]
    wef = We1[2 * IN + 1:]
    be1r = be1.reshape(1, -1)
    be2r = be2.reshape(1, -1)

    t1, t2 = _make_tables(node_feat, w1a, w1b)

    ms = []
    for sc in range(S):
        a, b, radial = _gather(sc, t1, t2, src, dst, cflat)
        m = _edge_mlp(sc, a, b, edge_feat, radial,
                      wr, wef, be1r, We2, be2r)
        ms.append(m)

    zeros = jnp.zeros((NPAD, HID), jnp.float32)
    part_a = _scatter(ms[:3], (0, 1, 2), dst, zeros)
    part_b = _scatter(ms[3:], (3, 4), dst, zeros)
    p0 = part_a[:N]
    p1 = part_a[NPAD:NPAD + N]
    p2 = part_b[:N]
    p3 = part_b[NPAD:NPAD + N]
    return _node_mlp(node_feat, p0, p1, p2, p3, Wn1[:IN], Wn1[IN:],
                     bn1.reshape(1, -1), Wn2, bn2.reshape(1, -1))
